# Initial kernel scaffold; baseline (speedup 1.0000x reference)
import jax
import jax.numpy as jnp
from jax import lax
from jax.experimental import pallas as pl
from jax.experimental.pallas import tpu as pltpu
from jax.experimental.pallas import tpu_sc as plsc

B, S, D = 4096, 200, 64
N = B * S
LW = 128
NW = 32
PER_W = N // NW
CHUNK = 512
KSUB = CHUNK // LW
NCH = PER_W // CHUNK
IDX_ROWS = N // LW


def _body(idx1, idx2, wq, ws, out1, out2, idx_v, rows_v, sem):
    cid = lax.axis_index("c")
    sid = lax.axis_index("s")
    wid = sid * 2 + cid

    def run_table(idx_hbm, tbl, out_hbm):
        def chunk(i, carry):
            r0 = wid * (PER_W // LW) + i * KSUB
            start = wid * PER_W + i * CHUNK
            pltpu.sync_copy(idx_hbm.at[pl.ds(r0, KSUB)], idx_v)
            cps = [
                pltpu.async_copy(
                    tbl.at[idx_v.at[j]],
                    rows_v.at[pl.ds(j * LW, LW)],
                    sem,
                )
                for j in range(KSUB)
            ]
            for cp in cps:
                cp.wait()
            pltpu.sync_copy(rows_v, out_hbm.at[pl.ds(start, CHUNK)])
            return carry

        lax.fori_loop(0, NCH, chunk, 0)

    run_table(idx1, wq, out1)
    run_table(idx2, ws, out2)


@jax.jit
def _emb(i1, i2, wq, ws):
    mesh = plsc.VectorSubcoreMesh(core_axis_name="c", subcore_axis_name="s")
    f = pl.kernel(
        _body,
        mesh=mesh,
        out_type=[
            jax.ShapeDtypeStruct((N, D), jnp.float32),
            jax.ShapeDtypeStruct((N, D), jnp.float32),
        ],
        scratch_types=[
            pltpu.VMEM((KSUB, LW), jnp.int32),
            pltpu.VMEM((CHUNK, D), jnp.float32),
            pltpu.SemaphoreType.DMA,
        ],
    )
    return f(i1, i2, wq, ws)


def kernel(input_1, input_2, Wq, Ws):
    wq = Wq.at[0].set(0.0)
    ws = Ws.at[0].set(0.0)
    i1 = input_1.reshape(IDX_ROWS, LW).astype(jnp.int32)
    i2 = input_2.reshape(IDX_ROWS, LW).astype(jnp.int32)
    o1, o2 = _emb(i1, i2, wq, ws)
    return o1.reshape(B, S, D), o2.reshape(B, S, D)


# speedup vs baseline: 4.1561x; 4.1561x over previous
import jax
import jax.numpy as jnp
from jax import lax
from jax.experimental import pallas as pl
from jax.experimental.pallas import tpu as pltpu
from jax.experimental.pallas import tpu_sc as plsc

B, S, D = 4096, 200, 64
N = B * S
LW = 128
NW = 32
PER_W = N // NW
CHUNK = 512
KSUB = CHUNK // LW
NCH = PER_W // CHUNK
IDX_ROWS = N // LW


def _body(idx1, idx2, wq, ws, out1, out2, idx_v, rows_v, sem):
    cid = lax.axis_index("c")
    sid = lax.axis_index("s")
    wid = sid * 2 + cid

    def run_table(idx_hbm, tbl, out_hbm):
        def chunk(i, carry):
            r0 = wid * (PER_W // LW) + i * KSUB
            start = wid * PER_W + i * CHUNK
            pltpu.sync_copy(idx_hbm.at[pl.ds(r0, KSUB)], idx_v)
            cps = [
                pltpu.async_copy(
                    tbl.at[idx_v.at[j]],
                    rows_v.at[pl.ds(j * LW, LW)],
                    sem,
                )
                for j in range(KSUB)
            ]
            for cp in cps:
                cp.wait()
            pltpu.sync_copy(rows_v, out_hbm.at[pl.ds(start, CHUNK)])
            return carry

        lax.fori_loop(0, NCH, chunk, 0)

    run_table(idx1, wq, out1)
    run_table(idx2, ws, out2)


@jax.jit
def _emb(i1, i2, wq, ws):
    mesh = plsc.VectorSubcoreMesh(core_axis_name="c", subcore_axis_name="s")
    f = pl.kernel(
        _body,
        mesh=mesh,
        out_type=[
            jax.ShapeDtypeStruct((N, D), jnp.float32),
            jax.ShapeDtypeStruct((N, D), jnp.float32),
        ],
        scratch_types=[
            pltpu.VMEM((KSUB, LW), jnp.int32),
            pltpu.VMEM((CHUNK, D), jnp.float32),
            pltpu.SemaphoreType.DMA,
        ],
        compiler_params=pltpu.CompilerParams(use_tc_tiling_on_sc=False),
    )
    return f(i1, i2, wq, ws)


def kernel(input_1, input_2, Wq, Ws):
    wq = Wq.at[0].set(0.0)
    ws = Ws.at[0].set(0.0)
    i1 = input_1.reshape(IDX_ROWS, LW).astype(jnp.int32)
    i2 = input_2.reshape(IDX_ROWS, LW).astype(jnp.int32)
    o1, o2 = _emb(i1, i2, wq, ws)
    return o1.reshape(B, S, D), o2.reshape(B, S, D)


# speedup vs baseline: 4.5000x; 1.0827x over previous
import jax
import jax.numpy as jnp
from jax import lax
from jax.experimental import pallas as pl
from jax.experimental.pallas import tpu as pltpu
from jax.experimental.pallas import tpu_sc as plsc

B, S, D = 4096, 200, 64
N = B * S
LW = 128
NW = 32
PER_W = N // NW
IROWS_W = PER_W // LW
CHUNK = 512
KSUB = CHUNK // LW
NCH = PER_W // CHUNK
NPAIR = NCH // 2
IDX_ROWS = N // LW


def _body(idx1, idx2, wq, ws, out1, out2, idx_v, rows0, rows1, semg0, semg1):
    cid = lax.axis_index("c")
    sid = lax.axis_index("s")
    wid = sid * 2 + cid
    rows = (rows0, rows1)
    semg = (semg0, semg1)

    def run_table(idx_hbm, tbl, out_hbm):
        base = wid * PER_W
        pltpu.sync_copy(idx_hbm.at[pl.ds(wid * IROWS_W, IROWS_W)], idx_v)

        def fire_g(i, b):
            for j in range(KSUB):
                pltpu.async_copy(
                    tbl.at[idx_v.at[i * KSUB + j]],
                    rows[b].at[pl.ds(j * LW, LW)],
                    semg[b],
                )

        def drain_g(b):
            pltpu.make_async_copy(tbl.at[pl.ds(0, CHUNK)], rows[b], semg[b]).wait()

        def wb(i, b):
            pltpu.sync_copy(rows[b], out_hbm.at[pl.ds(base + i * CHUNK, CHUNK)])

        fire_g(0, 0)

        def pair(g, carry):
            i = 2 * g
            drain_g(0)
            fire_g(i + 1, 1)
            wb(i, 0)
            drain_g(1)

            @pl.when(g < NPAIR - 1)
            def _():
                fire_g(i + 2, 0)

            wb(i + 1, 1)
            return carry

        lax.fori_loop(0, NPAIR, pair, 0)

    run_table(idx1, wq, out1)
    run_table(idx2, ws, out2)


@jax.jit
def _emb(i1, i2, wq, ws):
    mesh = plsc.VectorSubcoreMesh(core_axis_name="c", subcore_axis_name="s")
    f = pl.kernel(
        _body,
        mesh=mesh,
        out_type=[
            jax.ShapeDtypeStruct((N, D), jnp.float32),
            jax.ShapeDtypeStruct((N, D), jnp.float32),
        ],
        scratch_types=[
            pltpu.VMEM((IROWS_W, LW), jnp.int32),
            pltpu.VMEM((CHUNK, D), jnp.float32),
            pltpu.VMEM((CHUNK, D), jnp.float32),
            pltpu.SemaphoreType.DMA,
            pltpu.SemaphoreType.DMA,
        ],
        compiler_params=pltpu.CompilerParams(use_tc_tiling_on_sc=False),
    )
    return f(i1, i2, wq, ws)


def kernel(input_1, input_2, Wq, Ws):
    wq = Wq.at[0].set(0.0)
    ws = Ws.at[0].set(0.0)
    i1 = input_1.reshape(IDX_ROWS, LW).astype(jnp.int32)
    i2 = input_2.reshape(IDX_ROWS, LW).astype(jnp.int32)
    o1, o2 = _emb(i1, i2, wq, ws)
    return o1.reshape(B, S, D), o2.reshape(B, S, D)


# speedup vs baseline: 4.5076x; 1.0017x over previous
import jax
import jax.numpy as jnp
from jax import lax
from jax.experimental import pallas as pl
from jax.experimental.pallas import tpu as pltpu
from jax.experimental.pallas import tpu_sc as plsc

B, S, D = 4096, 200, 64
N = B * S
LW = 128
NW = 32
PER_W = N // NW
IROWS_W = PER_W // LW
CHUNK = 512
KSUB = CHUNK // LW
NCH = PER_W // CHUNK
NBUF = 3
NT = (NCH + NBUF - 1) // NBUF
IDX_ROWS = N // LW
CBYTES = CHUNK * D * 4


def _body(idx1, idx2, wq, ws, out1, out2, idx_v, r0, r1, r2,
          sg0, sg1, sg2, sw0, sw1, sw2):
    cid = lax.axis_index("c")
    sid = lax.axis_index("s")
    wid = sid * 2 + cid
    rows = (r0, r1, r2)
    semg = (sg0, sg1, sg2)
    semw = (sw0, sw1, sw2)

    def run_table(idx_hbm, tbl, out_hbm):
        base = wid * PER_W
        pltpu.sync_copy(idx_hbm.at[pl.ds(wid * IROWS_W, IROWS_W)], idx_v)

        def fire_g(i, b):
            for j in range(KSUB):
                pltpu.async_copy(
                    tbl.at[idx_v.at[i * KSUB + j]],
                    rows[b].at[pl.ds(j * LW, LW)],
                    semg[b],
                )

        def drain_g(b):
            pltpu.make_async_copy(tbl.at[pl.ds(0, CHUNK)], rows[b], semg[b]).wait()

        def wb_fire(i, b):
            pltpu.async_copy(rows[b], out_hbm.at[pl.ds(base + i * CHUNK, CHUNK)],
                             semw[b])

        def wb_wait(b):
            pltpu.make_async_copy(
                rows[b], out_hbm.at[pl.ds(0, CHUNK)], semw[b]).wait()

        fire_g(0, 0)
        fire_g(1, 1)

        def step(t, carry):
            for j in range(NBUF):
                i = NBUF * t + j
                b = j
                bn = (j + 2) % NBUF

                @pl.when(i < NCH)
                def _():
                    drain_g(b)
                    wb_fire(i, b)

                @pl.when(jnp.logical_and(i + 2 < NCH, i >= 1))
                def _():
                    wb_wait(bn)

                @pl.when(i + 2 < NCH)
                def _():
                    fire_g(i + 2, bn)

            return carry

        lax.fori_loop(0, NT, step, 0)
        for b in range(NBUF):
            wb_wait(b)

    run_table(idx1, wq, out1)
    run_table(idx2, ws, out2)


@jax.jit
def _emb(i1, i2, wq, ws):
    mesh = plsc.VectorSubcoreMesh(core_axis_name="c", subcore_axis_name="s")
    f = pl.kernel(
        _body,
        mesh=mesh,
        out_type=[
            jax.ShapeDtypeStruct((N, D), jnp.float32),
            jax.ShapeDtypeStruct((N, D), jnp.float32),
        ],
        scratch_types=[
            pltpu.VMEM((IROWS_W, LW), jnp.int32),
            pltpu.VMEM((CHUNK, D), jnp.float32),
            pltpu.VMEM((CHUNK, D), jnp.float32),
            pltpu.VMEM((CHUNK, D), jnp.float32),
            pltpu.SemaphoreType.DMA,
            pltpu.SemaphoreType.DMA,
            pltpu.SemaphoreType.DMA,
            pltpu.SemaphoreType.DMA,
            pltpu.SemaphoreType.DMA,
            pltpu.SemaphoreType.DMA,
        ],
        compiler_params=pltpu.CompilerParams(use_tc_tiling_on_sc=False),
    )
    return f(i1, i2, wq, ws)


def kernel(input_1, input_2, Wq, Ws):
    wq = Wq.at[0].set(0.0)
    ws = Ws.at[0].set(0.0)
    i1 = input_1.reshape(IDX_ROWS, LW).astype(jnp.int32)
    i2 = input_2.reshape(IDX_ROWS, LW).astype(jnp.int32)
    o1, o2 = _emb(i1, i2, wq, ws)
    return o1.reshape(B, S, D), o2.reshape(B, S, D)


# speedup vs baseline: 4.5077x; 1.0000x over previous
import jax
import jax.numpy as jnp
from jax import lax
from jax.experimental import pallas as pl
from jax.experimental.pallas import tpu as pltpu
from jax.experimental.pallas import tpu_sc as plsc

B, S, D = 4096, 200, 64
N = B * S
LW = 128
NW = 32
PER_W = N // NW
IROWS_W = PER_W // LW
CHUNK = 256
KSUB = CHUNK // LW
NCH = PER_W // CHUNK
NBUF = 5
K = 3
NT = (NCH + NBUF - 1) // NBUF
IDX_ROWS = N // LW


def _body(idx1, idx2, wq, ws, out1, out2, idx_v, r0, r1, r2, r3, r4,
          sg0, sg1, sg2, sg3, sg4, sw0, sw1, sw2, sw3, sw4):
    cid = lax.axis_index("c")
    sid = lax.axis_index("s")
    wid = sid * 2 + cid
    rows = (r0, r1, r2, r3, r4)
    semg = (sg0, sg1, sg2, sg3, sg4)
    semw = (sw0, sw1, sw2, sw3, sw4)

    def run_table(idx_hbm, tbl, out_hbm):
        base = wid * PER_W
        pltpu.sync_copy(idx_hbm.at[pl.ds(wid * IROWS_W, IROWS_W)], idx_v)

        def fire_g(i, b):
            for j in range(KSUB):
                pltpu.async_copy(
                    tbl.at[idx_v.at[i * KSUB + j]],
                    rows[b].at[pl.ds(j * LW, LW)],
                    semg[b],
                )

        def drain_g(b):
            pltpu.make_async_copy(tbl.at[pl.ds(0, CHUNK)], rows[b], semg[b]).wait()

        def wb_fire(i, b):
            pltpu.async_copy(rows[b], out_hbm.at[pl.ds(base + i * CHUNK, CHUNK)],
                             semw[b])

        def wb_wait(b):
            pltpu.make_async_copy(
                rows[b], out_hbm.at[pl.ds(0, CHUNK)], semw[b]).wait()

        for b in range(K):
            fire_g(b, b)

        def step(t, carry):
            for j in range(NBUF):
                i = NBUF * t + j
                b = j
                bf = (j + K) % NBUF

                @pl.when(i < NCH)
                def _():
                    drain_g(b)
                    wb_fire(i, b)

                @pl.when(jnp.logical_and(i + K < NCH, i >= NBUF - K))
                def _():
                    wb_wait(bf)

                @pl.when(i + K < NCH)
                def _():
                    fire_g(i + K, bf)

            return carry

        lax.fori_loop(0, NT, step, 0)
        for b in range(NBUF):
            wb_wait(b)

    run_table(idx1, wq, out1)
    run_table(idx2, ws, out2)


@jax.jit
def _emb(i1, i2, wq, ws):
    mesh = plsc.VectorSubcoreMesh(core_axis_name="c", subcore_axis_name="s")
    f = pl.kernel(
        _body,
        mesh=mesh,
        out_type=[
            jax.ShapeDtypeStruct((N, D), jnp.float32),
            jax.ShapeDtypeStruct((N, D), jnp.float32),
        ],
        scratch_types=[
            pltpu.VMEM((IROWS_W, LW), jnp.int32),
        ] + [pltpu.VMEM((CHUNK, D), jnp.float32)] * NBUF
          + [pltpu.SemaphoreType.DMA] * (2 * NBUF),
        compiler_params=pltpu.CompilerParams(use_tc_tiling_on_sc=False),
    )
    return f(i1, i2, wq, ws)


def kernel(input_1, input_2, Wq, Ws):
    wq = Wq.at[0].set(0.0)
    ws = Ws.at[0].set(0.0)
    i1 = input_1.reshape(IDX_ROWS, LW).astype(jnp.int32)
    i2 = input_2.reshape(IDX_ROWS, LW).astype(jnp.int32)
    o1, o2 = _emb(i1, i2, wq, ws)
    return o1.reshape(B, S, D), o2.reshape(B, S, D)
